# baseline (speedup 1.0000x reference)
import functools
import math

import jax
import jax.numpy as jnp
from jax.experimental import pallas as pl
from jax.experimental.pallas import tpu as pltpu

_SPARSITY = 0.8


def _reduce_body(x_ref, sum_ref, max_ref):
    f = pl.program_id(1)
    h = pl.program_id(2)
    xb = x_ref[0, 0]
    s = jnp.sum(xb, axis=-1)
    m = jnp.max(xb, axis=-1)

    @pl.when((f == 0) & (h == 0))
    def _():
        sum_ref[0, 0] = s
        max_ref[0, 0] = m

    @pl.when((f > 0) | (h > 0))
    def _():
        sum_ref[0, 0] = sum_ref[0, 0] + s
        max_ref[0, 0] = jnp.maximum(max_ref[0, 0], m)


def _scale_body(sum_ref, max_ref, w1_ref, w2_ref, scale_ref, *, n_red, k):
    avg = sum_ref[...][:, 0, :] * (1.0 / n_red)
    mx = max_ref[...][:, 0, :]
    w1 = w1_ref[...]
    w2 = w2_ref[...]

    def mlp(v):
        h = jnp.sum(v[:, None, :] * w1[None, :, :], axis=-1)
        h = jnp.maximum(h, 0.0)
        return jnp.sum(h[:, None, :] * w2[None, :, :], axis=-1)

    logit = mlp(avg) + mlp(mx)
    ca = 1.0 / (1.0 + jnp.exp(-logit))

    b, c = ca.shape
    sj = ca[:, None, :]
    si = ca[:, :, None]
    ii = jax.lax.broadcasted_iota(jnp.int32, (1, c, c), 1)
    jj = jax.lax.broadcasted_iota(jnp.int32, (1, c, c), 2)
    beats = (sj > si) | ((sj == si) & (jj < ii))
    rank = jnp.sum(beats.astype(jnp.int32), axis=-1)
    scale_ref[...] = jnp.where(rank < k, ca, 0.0)[:, None, :]


def _mul_body(x_ref, scale_ref, out_ref):
    out_ref[...] = x_ref[...] * scale_ref[0, 0][None, None, :, None]


def kernel(x, W1, W2):
    B, F, C, H, W = x.shape
    HW = H * W
    xr = x.reshape(B, F, C, HW)
    NHW = 1
    for cand in (8, 7, 4, 2):
        if HW % cand == 0 and (HW // cand) % 128 == 0:
            NHW = cand
            break
    HWT = HW // NHW
    k = int(math.ceil(C * _SPARSITY))

    grid = (B, F, NHW)
    x_spec = pl.BlockSpec((1, 1, C, HWT), lambda b, f, h: (b, f, 0, h))
    bc_spec = pl.BlockSpec((1, 1, C), lambda b, f, h: (b, 0, 0))

    sums, maxs = pl.pallas_call(
        _reduce_body,
        grid=grid,
        in_specs=[x_spec],
        out_specs=[bc_spec, bc_spec],
        out_shape=[jax.ShapeDtypeStruct((B, 1, C), jnp.float32)] * 2,
        compiler_params=pltpu.CompilerParams(
            dimension_semantics=("parallel", "arbitrary", "arbitrary")),
    )(xr)

    scale = pl.pallas_call(
        functools.partial(_scale_body, n_red=F * HW, k=k),
        out_shape=jax.ShapeDtypeStruct((B, 1, C), jnp.float32),
    )(sums, maxs, W1, W2)

    out = pl.pallas_call(
        _mul_body,
        grid=grid,
        in_specs=[x_spec, bc_spec],
        out_specs=x_spec,
        out_shape=jax.ShapeDtypeStruct((B, F, C, HW), jnp.float32),
        compiler_params=pltpu.CompilerParams(
            dimension_semantics=("parallel", "arbitrary", "arbitrary")),
    )(xr, scale)
    return out.reshape(B, F, C, H, W)
